# Initial kernel scaffold; baseline (speedup 1.0000x reference)
import jax
import jax.numpy as jnp
from jax.experimental import pallas as pl


def kernel(node_feature, edge_src, edge_dst):
    raise NotImplementedError("write your pallas kernel here")


# speedup vs baseline: 4.0113x; 4.0113x over previous
import functools

import jax
import jax.numpy as jnp
from jax import lax
from jax.experimental import pallas as pl
from jax.experimental.pallas import tpu as pltpu
from jax.experimental.pallas import tpu_sc as plsc

N_NODES = 10000
N_EDGES = 320000
D_FEAT = 128
LANES = 16

NUM_CORES = 2
NUM_SUBCORES = 16
NUM_WORKERS = NUM_CORES * NUM_SUBCORES
EDGES_PER_WORKER = N_EDGES // NUM_WORKERS
CHUNK = 400
NUM_CHUNKS = EDGES_PER_WORKER // CHUNK


def _sc_body(feat_hbm, src_hbm, dst_hbm, out_hbm,
             sidx_v, didx_v, srows_v, drows_v, outv, sem_s, sem_d):
    wid = lax.axis_index("s") * NUM_CORES + lax.axis_index("c")
    base_w = wid * EDGES_PER_WORKER

    lane_iota = lax.iota(jnp.int32, LANES)
    rots = [(lane_iota + t) % LANES for t in range(LANES)]

    def chunk_body(i, carry):
        base = base_w + i * CHUNK
        pltpu.sync_copy(src_hbm.at[pl.ds(base, CHUNK)], sidx_v)
        pltpu.sync_copy(dst_hbm.at[pl.ds(base, CHUNK)], didx_v)
        cp_s = pltpu.async_copy(feat_hbm.at[sidx_v], srows_v, sem_s)
        cp_d = pltpu.async_copy(feat_hbm.at[didx_v], drows_v, sem_d)
        cp_s.wait()
        cp_d.wait()

        @plsc.parallel_loop(0, CHUNK // LANES, 1, unroll=2)
        def group_body(g):
            e0 = g * LANES
            rows = lane_iota + e0
            res = jnp.zeros((LANES,), jnp.float32)
            for blk in range(D_FEAT // LANES):
                for t in range(LANES):
                    cols = rots[t] + (blk * LANES)
                    sv = plsc.load_gather(srows_v, [rows, cols])
                    dv = plsc.load_gather(drows_v, [rows, cols])
                    res = res + sv * dv
            outv[pl.ds(e0, LANES)] = res

        pltpu.sync_copy(outv, out_hbm.at[pl.ds(base, CHUNK)])
        return carry

    lax.fori_loop(0, NUM_CHUNKS, chunk_body, 0, unroll=False)


@jax.jit
def _edge_dot(node_feature, src_i32, dst_i32):
    mesh = plsc.VectorSubcoreMesh(core_axis_name="c", subcore_axis_name="s")
    scores = pl.kernel(
        _sc_body,
        out_type=jax.ShapeDtypeStruct((N_EDGES,), jnp.float32),
        mesh=mesh,
        compiler_params=pltpu.CompilerParams(needs_layout_passes=False),
        scratch_types=[
            pltpu.VMEM((CHUNK,), jnp.int32),
            pltpu.VMEM((CHUNK,), jnp.int32),
            pltpu.VMEM((CHUNK, D_FEAT), jnp.float32),
            pltpu.VMEM((CHUNK, D_FEAT), jnp.float32),
            pltpu.VMEM((CHUNK,), jnp.float32),
            pltpu.SemaphoreType.DMA,
            pltpu.SemaphoreType.DMA,
        ],
    )(node_feature, src_i32, dst_i32)
    return scores.reshape(N_EDGES, 1)


def kernel(node_feature, edge_src, edge_dst):
    src_i32 = edge_src.astype(jnp.int32)
    dst_i32 = edge_dst.astype(jnp.int32)
    return _edge_dot(node_feature, src_i32, dst_i32)
